# baseline (speedup 1.0000x reference)
import functools

import jax
import jax.numpy as jnp
from jax import lax
from jax.experimental import pallas as pl
from jax.experimental.pallas import tpu as pltpu
from jax.experimental.pallas import tpu_sc as plsc

N = 10000
E = 320000
F = 128

NC = 2
NS = 16
NW = NC * NS

CHUNK = 128
CH3 = 128
NPAD = 10240
DEG_SLICE = NPAD // NS
ROWS_T = NPAD // NS
GPW = 80
GPW3 = 80
E_PAD = 2624 * CHUNK

_mesh = plsc.VectorSubcoreMesh(core_axis_name="c", subcore_axis_name="s")


@functools.partial(
    pl.kernel,
    out_type=jax.ShapeDtypeStruct((NC * NPAD,), jnp.float32),
    mesh=_mesh,
    scratch_types=[
        [pltpu.VMEM((CHUNK,), jnp.int32) for _ in range(4)],
        pltpu.VMEM((CHUNK,), jnp.float32),
        pltpu.VMEM((DEG_SLICE,), jnp.float32),
        pltpu.VMEM_SHARED((NPAD,), jnp.float32),
        [pltpu.SemaphoreType.DMA for _ in range(4)],
        [pltpu.SemaphoreType.DMA for _ in range(4)],
    ],
)
def _deg_kernel(row_hbm, zeros_hbm, ones_hbm, deg_hbm,
                I, ones_v, zv, acc, SI, SS):
    c = lax.axis_index("c")
    s = lax.axis_index("s")
    w = c * NS + s

    def start_idx(g, q):
        pltpu.async_copy(row_hbm.at[pl.ds((w + g * NW) * CHUNK, CHUNK)],
                         I[q], SI[q])

    def wait_idx(g, q):
        pltpu.make_async_copy(row_hbm.at[pl.ds((w + g * NW) * CHUNK, CHUNK)],
                              I[q], SI[q]).wait()

    def wait_scatter(q):
        pltpu.make_async_copy(ones_v, acc.at[I[q]], SS[q]).wait()

    pltpu.sync_copy(zeros_hbm, zv)
    pltpu.sync_copy(zv, acc.at[pl.ds(s * DEG_SLICE, DEG_SLICE)])
    pltpu.sync_copy(ones_hbm, ones_v)
    plsc.subcore_barrier()

    start_idx(0, 0)
    start_idx(1, 1)
    pltpu.async_copy(ones_hbm, ones_v, SS[2])
    pltpu.async_copy(ones_hbm, ones_v, SS[3])

    def body(j, _):
        g0 = j * 4
        for q in range(4):
            g = g0 + q
            q2 = (q + 2) % 4
            wait_idx(g, q)
            pltpu.async_copy(ones_v, acc.at[I[q]], SS[q], add=True)
            wait_scatter(q2)
            start_idx(g + 2, q2)
        return 0

    lax.fori_loop(0, GPW // 4, body, 0)
    wait_scatter(2)
    wait_scatter(3)
    wait_idx(GPW, 0)
    wait_idx(GPW + 1, 1)
    plsc.subcore_barrier()

    pltpu.sync_copy(acc.at[pl.ds(s * DEG_SLICE, DEG_SLICE)], zv)
    pltpu.sync_copy(zv, deg_hbm.at[pl.ds(c * NPAD + s * DEG_SLICE, DEG_SLICE)])


@functools.partial(
    pl.kernel,
    out_type=jax.ShapeDtypeStruct((NC, NPAD, F), jnp.float32),
    mesh=_mesh,
    scratch_types=[
        [pltpu.VMEM((CH3,), jnp.int32) for _ in range(2)],
        [pltpu.VMEM((CH3,), jnp.int32) for _ in range(2)],
        [pltpu.VMEM((CH3, F), jnp.float32) for _ in range(2)],
        pltpu.VMEM_SHARED((NPAD, F), jnp.float32),
        [pltpu.SemaphoreType.DMA for _ in range(2)],
        [pltpu.SemaphoreType.DMA for _ in range(2)],
        [pltpu.SemaphoreType.DMA for _ in range(2)],
    ],
)
def _seg_kernel(xs_hbm, row_hbm, col_hbm, zblk_hbm, s_hbm,
                RI, CI, RV, acc, SR, SC, SG):
    c = lax.axis_index("c")
    s = lax.axis_index("s")
    w = c * NS + s

    def base(g):
        return (w + g * NW) * CH3

    def start_idx(g, q):
        pltpu.async_copy(row_hbm.at[pl.ds(base(g), CH3)], RI[q], SR[q])
        pltpu.async_copy(col_hbm.at[pl.ds(base(g), CH3)], CI[q], SC[q])

    def wait_row_idx(g, q):
        pltpu.make_async_copy(row_hbm.at[pl.ds(base(g), CH3)], RI[q], SR[q]).wait()

    def wait_col_idx(g, q):
        pltpu.make_async_copy(col_hbm.at[pl.ds(base(g), CH3)], CI[q], SC[q]).wait()

    def start_gather(q):
        pltpu.async_copy(xs_hbm.at[RI[q]], RV[q], SG[q])

    def wait_gather(q):
        pltpu.make_async_copy(xs_hbm.at[RI[q]], RV[q], SG[q]).wait()

    pltpu.sync_copy(zblk_hbm, RV[0])
    for j in range(ROWS_T // CH3):
        pltpu.sync_copy(RV[0], acc.at[pl.ds(s * ROWS_T + j * CH3, CH3)])
    plsc.subcore_barrier()

    start_idx(0, 0)
    wait_row_idx(0, 0)
    start_gather(0)
    start_idx(1, 1)

    def body(j, _):
        g0 = j * 2
        for p in range(2):
            g = g0 + p
            wait_row_idx(g + 1, 1 - p)
            start_gather(1 - p)
            wait_gather(p)
            pltpu.async_copy(row_hbm.at[pl.ds(base(g + 2), CH3)], RI[p], SR[p])
            wait_col_idx(g, p)
            pltpu.sync_copy(RV[p], acc.at[CI[p]], add=True)
            pltpu.async_copy(col_hbm.at[pl.ds(base(g + 2), CH3)], CI[p], SC[p])
        return 0

    lax.fori_loop(0, GPW3 // 2, body, 0)

    wait_row_idx(GPW3 + 1, 1)
    wait_col_idx(GPW3 + 1, 1)
    wait_gather(0)
    wait_col_idx(GPW3, 0)
    plsc.subcore_barrier()

    pltpu.sync_copy(acc.at[pl.ds(s * ROWS_T, ROWS_T)],
                    s_hbm.at[c, pl.ds(s * ROWS_T, ROWS_T)])


_RB = 400


def _k2a_body(x_ref, w0_ref, w1_ref, b_ref, y1_ref, z0_ref):
    x = x_ref[...]
    y1_ref[...] = jnp.dot(x, w1_ref[...], preferred_element_type=jnp.float32)
    z0_ref[...] = jnp.dot(x, w0_ref[...], preferred_element_type=jnp.float32) + b_ref[...]


def _k2b_body(y1_ref, dega_ref, degb_ref, xs_ref, dinv_ref):
    deg = dega_ref[...] + degb_ref[...]
    dinv = jnp.where(deg > 0, lax.rsqrt(deg), 0.0)
    xs_ref[...] = dinv * y1_ref[...]
    dinv_ref[...] = dinv


def _k4_body(z0_ref, dinv_ref, s_ref, o_ref):
    stot = s_ref[0] + s_ref[1]
    o_ref[...] = jnp.maximum(z0_ref[...] - dinv_ref[...] * stot, 0.0)


def kernel(x, adj, W0, W1, b):
    row = adj[0]
    col = adj[1]
    pad_i = jnp.arange(E_PAD - E, dtype=jnp.int32)
    dummy = N + pad_i % (NPAD - N)
    row_g = jnp.concatenate([row, pad_i % N])
    col_s = jnp.concatenate([col, dummy])
    row_d = jnp.concatenate([row, dummy])
    zeros_deg = jnp.zeros((DEG_SLICE,), jnp.float32)
    ones_chunk = jnp.ones((CHUNK,), jnp.float32)
    zblk = jnp.zeros((CH3, F), jnp.float32)

    deg = _deg_kernel(row_d, zeros_deg, ones_chunk)
    dega = deg[:N, None]
    degb = deg[NPAD:NPAD + N, None]

    y1, z0 = pl.pallas_call(
        _k2a_body,
        grid=(N // _RB,),
        in_specs=[
            pl.BlockSpec((_RB, F), lambda i: (i, 0)),
            pl.BlockSpec((F, F), lambda i: (0, 0)),
            pl.BlockSpec((F, F), lambda i: (0, 0)),
            pl.BlockSpec((1, F), lambda i: (0, 0)),
        ],
        out_specs=[
            pl.BlockSpec((_RB, F), lambda i: (i, 0)),
            pl.BlockSpec((_RB, F), lambda i: (i, 0)),
        ],
        out_shape=[
            jax.ShapeDtypeStruct((N, F), jnp.float32),
            jax.ShapeDtypeStruct((N, F), jnp.float32),
        ],
    )(x, W0, W1, b.reshape(1, F))

    xs, dinv = pl.pallas_call(
        _k2b_body,
        grid=(N // _RB,),
        in_specs=[
            pl.BlockSpec((_RB, F), lambda i: (i, 0)),
            pl.BlockSpec((_RB, 1), lambda i: (i, 0)),
            pl.BlockSpec((_RB, 1), lambda i: (i, 0)),
        ],
        out_specs=[
            pl.BlockSpec((_RB, F), lambda i: (i, 0)),
            pl.BlockSpec((_RB, 1), lambda i: (i, 0)),
        ],
        out_shape=[
            jax.ShapeDtypeStruct((N, F), jnp.float32),
            jax.ShapeDtypeStruct((N, 1), jnp.float32),
        ],
    )(y1, dega, degb)

    S = _seg_kernel(xs, row_g, col_s, zblk)

    out = pl.pallas_call(
        _k4_body,
        grid=(N // _RB,),
        in_specs=[
            pl.BlockSpec((_RB, F), lambda i: (i, 0)),
            pl.BlockSpec((_RB, 1), lambda i: (i, 0)),
            pl.BlockSpec((NC, _RB, F), lambda i: (0, i, 0)),
        ],
        out_specs=pl.BlockSpec((_RB, F), lambda i: (i, 0)),
        out_shape=jax.ShapeDtypeStruct((N, F), jnp.float32),
    )(z0, dinv, S)
    return out


# speedup vs baseline: 1.0609x; 1.0609x over previous
import functools

import jax
import jax.numpy as jnp
from jax import lax
from jax.experimental import pallas as pl
from jax.experimental.pallas import tpu as pltpu
from jax.experimental.pallas import tpu_sc as plsc

N = 10000
E = 320000
F = 128

NC = 2
NS = 16
NW = NC * NS

CHUNK = 128
CH3 = 128
NPAD = 10240
DEG_SLICE = NPAD // NS
ROWS_T = NPAD // NS
GPW = 80
GPW3 = 80
E_PAD = 2752 * CHUNK

_mesh = plsc.VectorSubcoreMesh(core_axis_name="c", subcore_axis_name="s")


@functools.partial(
    pl.kernel,
    out_type=jax.ShapeDtypeStruct((NC * NPAD,), jnp.float32),
    mesh=_mesh,
    scratch_types=[
        [pltpu.VMEM((CHUNK,), jnp.int32) for _ in range(8)],
        pltpu.VMEM((CHUNK,), jnp.float32),
        pltpu.VMEM((DEG_SLICE,), jnp.float32),
        pltpu.VMEM_SHARED((NPAD,), jnp.float32),
        [pltpu.SemaphoreType.DMA for _ in range(8)],
        [pltpu.SemaphoreType.DMA for _ in range(8)],
    ],
)
def _deg_kernel(row_hbm, zeros_hbm, ones_hbm, deg_hbm,
                I, ones_v, zv, acc, SI, SS):
    c = lax.axis_index("c")
    s = lax.axis_index("s")
    w = c * NS + s

    def start_idx(g, q):
        pltpu.async_copy(row_hbm.at[pl.ds((w + g * NW) * CHUNK, CHUNK)],
                         I[q], SI[q])

    def wait_idx(g, q):
        pltpu.make_async_copy(row_hbm.at[pl.ds((w + g * NW) * CHUNK, CHUNK)],
                              I[q], SI[q]).wait()

    def wait_scatter(q):
        pltpu.make_async_copy(ones_v, acc.at[I[q]], SS[q]).wait()

    pltpu.sync_copy(zeros_hbm, zv)
    pltpu.sync_copy(zv, acc.at[pl.ds(s * DEG_SLICE, DEG_SLICE)])
    pltpu.sync_copy(ones_hbm, ones_v)
    plsc.subcore_barrier()

    for q in range(6):
        start_idx(q, q)
    pltpu.async_copy(ones_hbm, ones_v, SS[6])
    pltpu.async_copy(ones_hbm, ones_v, SS[7])

    def body(j, _):
        g0 = j * 8
        for q in range(8):
            g = g0 + q
            q6 = (q + 6) % 8
            wait_idx(g, q)
            pltpu.async_copy(ones_v, acc.at[I[q]], SS[q], add=True)
            wait_scatter(q6)
            start_idx(g + 6, q6)
        return 0

    lax.fori_loop(0, GPW // 8, body, 0)
    wait_scatter(6)
    wait_scatter(7)
    for q in range(6):
        wait_idx(GPW + q, q)
    plsc.subcore_barrier()

    pltpu.sync_copy(acc.at[pl.ds(s * DEG_SLICE, DEG_SLICE)], zv)
    pltpu.sync_copy(zv, deg_hbm.at[pl.ds(c * NPAD + s * DEG_SLICE, DEG_SLICE)])


@functools.partial(
    pl.kernel,
    out_type=jax.ShapeDtypeStruct((NC, NPAD, F), jnp.float32),
    mesh=_mesh,
    scratch_types=[
        [pltpu.VMEM((CH3,), jnp.int32) for _ in range(2)],
        [pltpu.VMEM((CH3,), jnp.int32) for _ in range(2)],
        [pltpu.VMEM((CH3, F), jnp.float32) for _ in range(2)],
        pltpu.VMEM_SHARED((NPAD, F), jnp.float32),
        [pltpu.SemaphoreType.DMA for _ in range(2)],
        [pltpu.SemaphoreType.DMA for _ in range(2)],
        [pltpu.SemaphoreType.DMA for _ in range(2)],
    ],
)
def _seg_kernel(xs_hbm, row_hbm, col_hbm, zblk_hbm, s_hbm,
                RI, CI, RV, acc, SR, SC, SG):
    c = lax.axis_index("c")
    s = lax.axis_index("s")
    w = c * NS + s

    def base(g):
        return (w + g * NW) * CH3

    def start_idx(g, q):
        pltpu.async_copy(row_hbm.at[pl.ds(base(g), CH3)], RI[q], SR[q])
        pltpu.async_copy(col_hbm.at[pl.ds(base(g), CH3)], CI[q], SC[q])

    def wait_row_idx(g, q):
        pltpu.make_async_copy(row_hbm.at[pl.ds(base(g), CH3)], RI[q], SR[q]).wait()

    def wait_col_idx(g, q):
        pltpu.make_async_copy(col_hbm.at[pl.ds(base(g), CH3)], CI[q], SC[q]).wait()

    def start_gather(q):
        pltpu.async_copy(xs_hbm.at[RI[q]], RV[q], SG[q])

    def wait_gather(q):
        pltpu.make_async_copy(xs_hbm.at[RI[q]], RV[q], SG[q]).wait()

    pltpu.sync_copy(zblk_hbm, RV[0])
    for j in range(ROWS_T // CH3):
        pltpu.sync_copy(RV[0], acc.at[pl.ds(s * ROWS_T + j * CH3, CH3)])
    plsc.subcore_barrier()

    start_idx(0, 0)
    wait_row_idx(0, 0)
    start_gather(0)
    start_idx(1, 1)

    def body(j, _):
        g0 = j * 2
        for p in range(2):
            g = g0 + p
            wait_row_idx(g + 1, 1 - p)
            start_gather(1 - p)
            wait_gather(p)
            pltpu.async_copy(row_hbm.at[pl.ds(base(g + 2), CH3)], RI[p], SR[p])
            wait_col_idx(g, p)
            pltpu.sync_copy(RV[p], acc.at[CI[p]], add=True)
            pltpu.async_copy(col_hbm.at[pl.ds(base(g + 2), CH3)], CI[p], SC[p])
        return 0

    lax.fori_loop(0, GPW3 // 2, body, 0)

    wait_row_idx(GPW3 + 1, 1)
    wait_col_idx(GPW3 + 1, 1)
    wait_gather(0)
    wait_col_idx(GPW3, 0)
    plsc.subcore_barrier()

    pltpu.sync_copy(acc.at[pl.ds(s * ROWS_T, ROWS_T)],
                    s_hbm.at[c, pl.ds(s * ROWS_T, ROWS_T)])


_RB = 400


def _k2_body(x_ref, dega_ref, degb_ref, w0_ref, w1_ref, b_ref,
             xs_ref, z0_ref, dinv_ref):
    x = x_ref[...]
    deg = dega_ref[...] + degb_ref[...]
    dinv = jnp.where(deg > 0, lax.rsqrt(deg), 0.0)
    xs_ref[...] = dinv * jnp.dot(x, w1_ref[...], preferred_element_type=jnp.float32)
    z0_ref[...] = jnp.dot(x, w0_ref[...], preferred_element_type=jnp.float32) + b_ref[...]
    dinv_ref[...] = dinv


def _k4_body(z0_ref, dinv_ref, s_ref, o_ref):
    stot = s_ref[0] + s_ref[1]
    o_ref[...] = jnp.maximum(z0_ref[...] - dinv_ref[...] * stot, 0.0)


def kernel(x, adj, W0, W1, b):
    row = adj[0]
    col = adj[1]
    pad_i = jnp.arange(E_PAD - E, dtype=jnp.int32)
    dummy = N + pad_i % (NPAD - N)
    row_g = jnp.concatenate([row, pad_i % N])
    col_s = jnp.concatenate([col, dummy])
    row_d = jnp.concatenate([row, dummy])
    zeros_deg = jnp.zeros((DEG_SLICE,), jnp.float32)
    ones_chunk = jnp.ones((CHUNK,), jnp.float32)
    zblk = jnp.zeros((CH3, F), jnp.float32)

    deg = _deg_kernel(row_d, zeros_deg, ones_chunk)
    dega = deg[:N, None]
    degb = deg[NPAD:NPAD + N, None]

    xs, z0, dinv = pl.pallas_call(
        _k2_body,
        grid=(N // _RB,),
        in_specs=[
            pl.BlockSpec((_RB, F), lambda i: (i, 0)),
            pl.BlockSpec((_RB, 1), lambda i: (i, 0)),
            pl.BlockSpec((_RB, 1), lambda i: (i, 0)),
            pl.BlockSpec((F, F), lambda i: (0, 0)),
            pl.BlockSpec((F, F), lambda i: (0, 0)),
            pl.BlockSpec((1, F), lambda i: (0, 0)),
        ],
        out_specs=[
            pl.BlockSpec((_RB, F), lambda i: (i, 0)),
            pl.BlockSpec((_RB, F), lambda i: (i, 0)),
            pl.BlockSpec((_RB, 1), lambda i: (i, 0)),
        ],
        out_shape=[
            jax.ShapeDtypeStruct((N, F), jnp.float32),
            jax.ShapeDtypeStruct((N, F), jnp.float32),
            jax.ShapeDtypeStruct((N, 1), jnp.float32),
        ],
    )(x, dega, degb, W0, W1, b.reshape(1, F))

    S = _seg_kernel(xs, row_g, col_s, zblk)

    out = pl.pallas_call(
        _k4_body,
        grid=(N // _RB,),
        in_specs=[
            pl.BlockSpec((_RB, F), lambda i: (i, 0)),
            pl.BlockSpec((_RB, 1), lambda i: (i, 0)),
            pl.BlockSpec((NC, _RB, F), lambda i: (0, i, 0)),
        ],
        out_specs=pl.BlockSpec((_RB, F), lambda i: (i, 0)),
        out_shape=jax.ShapeDtypeStruct((N, F), jnp.float32),
    )(z0, dinv, S)
    return out
